# baseline (speedup 1.0000x reference)
import jax
import jax.numpy as jnp
from jax import lax
from jax.experimental import pallas as pl
from jax.experimental.pallas import tpu as pltpu
from jax.experimental.pallas import tpu_sc as plsc

_BATCH = 1024
_V = 200
_U1 = 101
_NORM = 18.8736
_L = 16
_NCHUNK = _BATCH // _L
_NWORK = 32
_CPW = _NCHUNK // _NWORK
_PEN = -1.0e38


def _sc_body(s_hbm, t_hbm, seq_hbm, size_hbm,
             sv0, sv1, tv, matched, seqbuf, sizebuf, sm0, sm1):
    wid = lax.axis_index("s") * 2 + lax.axis_index("c")
    lane = lax.broadcasted_iota(jnp.int32, (_L,), 0)
    zeros = jnp.zeros((_L,), jnp.float32)
    pen = jnp.full((_L,), _PEN, jnp.float32)

    def chunk_body(k, _):
        c = wid * _CPW + k
        for u in range(_U1):
            matched[u] = zeros
        pltpu.sync_copy(t_hbm.at[c], tv)
        pltpu.make_async_copy(s_hbm.at[c, 0], sv0, sm0).start()
        pltpu.make_async_copy(s_hbm.at[c, 1], sv1, sm1).start()

        def v_body(i, size):
            for b in range(2):
                v = 2 * i + b
                sm = (sm0, sm1)[b]
                sv = (sv0, sv1)[b]
                pltpu.make_async_copy(s_hbm.at[c, v], sv, sm).wait()
                t = tv[...]
                best = jnp.full((_L,), -1e30, jnp.float32)
                bidx = jnp.zeros((_L,), jnp.int32)
                for u in range(1, _U1):
                    wu = sv[u]
                    gu = sv[_U1 + u]
                    gm = gu + matched[u]
                    setm = (wu * _NORM + 1.0) >= t
                    upd = setm & (gm > best)
                    best = jnp.where(upd, gm, best)
                    bidx = jnp.where(upd, jnp.int32(u), bidx)
                anyv = best > -1e29
                sel = jnp.where(anyv, bidx, jnp.int32(0))
                wsel = plsc.load_gather(sv, [sel, lane])
                size = size + jnp.where(anyv, wsel, 0.0)
                plsc.store_scatter(matched, [sel, lane], pen, mask=anyv)
                plsc.store_scatter(seqbuf, [jnp.full((_L,), v, jnp.int32), lane], sel)
                nv = v + 2

                @pl.when(nv < _V)
                def _():
                    pltpu.make_async_copy(s_hbm.at[c, nv], sv, sm).start()
            return size

        size = lax.fori_loop(0, _V // 2, v_body, zeros)
        sizebuf[...] = -size
        pltpu.sync_copy(seqbuf, seq_hbm.at[c])
        pltpu.sync_copy(sizebuf, size_hbm.at[c])
        return 0

    lax.fori_loop(0, _CPW, chunk_body, 0)


@jax.jit
def _sc_call(s_t, t_t):
    mesh = plsc.VectorSubcoreMesh(core_axis_name="c", subcore_axis_name="s")
    f = pl.kernel(
        _sc_body,
        out_type=(
            jax.ShapeDtypeStruct((_NCHUNK, _V, _L), jnp.int32),
            jax.ShapeDtypeStruct((_NCHUNK, _L), jnp.float32),
        ),
        mesh=mesh,
        scratch_types=[
            pltpu.VMEM((2 * _U1, _L), jnp.float32),
            pltpu.VMEM((2 * _U1, _L), jnp.float32),
            pltpu.VMEM((_L,), jnp.float32),
            pltpu.VMEM((_U1, _L), jnp.float32),
            pltpu.VMEM((_V, _L), jnp.int32),
            pltpu.VMEM((_L,), jnp.float32),
            pltpu.SemaphoreType.DMA,
            pltpu.SemaphoreType.DMA,
        ],
        compiler_params=pltpu.CompilerParams(needs_layout_passes=False),
    )
    return f(s_t, t_t)


def kernel(weights):
    kt, kg = jax.random.split(jax.random.key(42))
    t = jnp.exp(jax.random.randint(kt, (_BATCH, 1), 1, 3).astype(jnp.float32))
    gumbel = jax.random.gumbel(kg, (_V, _BATCH, _U1), dtype=jnp.float32)
    w_t = weights.reshape(_NCHUNK, _L, _V, _U1).transpose(0, 2, 3, 1)
    g_t = gumbel.reshape(_V, _NCHUNK, _L, _U1).transpose(1, 0, 3, 2)
    s_t = jnp.concatenate([w_t, g_t], axis=2)
    t_t = t.reshape(_NCHUNK, _L)
    seq, neg_size = _sc_call(s_t, t_t)
    return neg_size.reshape(_BATCH), seq.transpose(0, 2, 1).reshape(_BATCH, _V)
